# Initial kernel scaffold; baseline (speedup 1.0000x reference)
import jax
import jax.numpy as jnp
from jax.experimental import pallas as pl


def kernel(h, edge_index, r, norm, W1, loop_w1, bias1, W2, loop_w2, bias2):
    raise NotImplementedError("write your pallas kernel here")


# speedup vs baseline: 23.5316x; 23.5316x over previous
import functools

import jax
import jax.numpy as jnp
from jax import lax
from jax.experimental import pallas as pl
from jax.experimental.pallas import tpu as pltpu
from jax.experimental.pallas import tpu_sc as plsc

_N = 10000
_E = 320000
_D = 128
_R = 16
_B = 8
_SUB = _D // _B

_NC = 2
_NS = 16
_NW = _NC * _NS
_EPW = _E // _NW
_KB = 80
_SB = 25
_NSB = _EPW // (_SB * _KB)
_NP = 10240
_STRIPE = _NP // _NS
_TN = 1000
_NT = _N // _TN


def _expand_weights(W, loop_w):
    Wb = W.reshape(_R, _B, _SUB, _SUB)
    eye = jnp.eye(_B, dtype=W.dtype)
    Wfull = jnp.einsum('rbio,bc->rbico', Wb, eye).reshape(_R, _D, _D)
    return jnp.concatenate([Wfull, loop_w[None]], axis=0)


def _mm_body(h_ref, w_ref, out_ref):
    out_ref[0] = jnp.dot(h_ref[...], w_ref[0], preferred_element_type=jnp.float32)


def _table_from_h(h, Wcat):
    return pl.pallas_call(
        _mm_body,
        grid=(_NT, _R + 1),
        in_specs=[
            pl.BlockSpec((_TN, _D), lambda i, j: (i, 0)),
            pl.BlockSpec((1, _D, _D), lambda i, j: (j, 0, 0)),
        ],
        out_specs=pl.BlockSpec((1, _TN, _D), lambda i, j: (j, i, 0)),
        out_shape=jax.ShapeDtypeStruct((_R + 1, _N, _D), jnp.float32),
    )(h, Wcat)


def _fuse_body(agg_ref, lp_ref, b_ref, w_ref, out_ref, h1_ref):
    @pl.when(pl.program_id(1) == 0)
    def _():
        h1_ref[...] = jnp.maximum(
            agg_ref[0] + agg_ref[1] + lp_ref[0] + b_ref[...], 0.0)

    out_ref[0] = jnp.dot(h1_ref[...], w_ref[0], preferred_element_type=jnp.float32)


def _table_from_agg(agg, tmp_prev, bias, Wcat):
    return pl.pallas_call(
        _fuse_body,
        grid=(_NT, _R + 1),
        in_specs=[
            pl.BlockSpec((2, _TN, _D), lambda i, j: (0, i, 0)),
            pl.BlockSpec((1, _TN, _D), lambda i, j: (_R, i, 0)),
            pl.BlockSpec((1, _D), lambda i, j: (0, 0)),
            pl.BlockSpec((1, _D, _D), lambda i, j: (j, 0, 0)),
        ],
        out_specs=pl.BlockSpec((1, _TN, _D), lambda i, j: (j, i, 0)),
        out_shape=jax.ShapeDtypeStruct((_R + 1, _N, _D), jnp.float32),
        scratch_shapes=[pltpu.VMEM((_TN, _D), jnp.float32)],
    )(agg, tmp_prev, bias.reshape(1, _D), Wcat)


def _final_body(agg_ref, lp_ref, b_ref, out_ref):
    out_ref[...] = agg_ref[0] + agg_ref[1] + lp_ref[0] + b_ref[...]


def _final(agg, tmp_prev, bias):
    return pl.pallas_call(
        _final_body,
        grid=(_NT,),
        in_specs=[
            pl.BlockSpec((2, _TN, _D), lambda i: (0, i, 0)),
            pl.BlockSpec((1, _TN, _D), lambda i: (_R, i, 0)),
            pl.BlockSpec((1, _D), lambda i: (0, 0)),
        ],
        out_specs=pl.BlockSpec((_TN, _D), lambda i: (i, 0)),
        out_shape=jax.ShapeDtypeStruct((_N, _D), jnp.float32),
    )(agg, tmp_prev, bias.reshape(1, _D))


def _sc_gather_scatter(table2d, src3, r3, dst3, norm3, zeros):
    mesh = plsc.VectorSubcoreMesh(core_axis_name="c", subcore_axis_name="s")

    @functools.partial(
        pl.kernel,
        out_type=jax.ShapeDtypeStruct((_NC, _NP, _D), jnp.float32),
        mesh=mesh,
        scratch_types=[
            pltpu.VMEM((_SB, _KB), jnp.int32),
            pltpu.VMEM((_SB, _KB), jnp.int32),
            pltpu.VMEM((_SB, _KB), jnp.int32),
            pltpu.VMEM((_SB, _KB), jnp.float32),
            pltpu.VMEM((2, _KB, _D), jnp.float32),
            pltpu.VMEM_SHARED((_NP, _D), jnp.float32),
            pltpu.SemaphoreType.DMA,
            pltpu.SemaphoreType.DMA,
        ],
    )
    def k(table_hbm, src_hbm, r_hbm, dst_hbm, norm_hbm, zero_hbm, out_hbm,
          gidx_v, r_v, dst_v, norm_v, rows_v, accum_sh, sem0, sem1):
        c = lax.axis_index("c")
        s = lax.axis_index("s")
        wid = c * _NS + s

        pltpu.sync_copy(zero_hbm.at[pl.ds(s * _STRIPE, _STRIPE)],
                        accum_sh.at[pl.ds(s * _STRIPE, _STRIPE)])
        plsc.subcore_barrier()

        def _gather_start(j, buf, sem):
            return pltpu.async_copy(table_hbm.at[gidx_v.at[j]], rows_v.at[buf], sem)

        def _gather_wait(j, buf, sem):
            pltpu.make_async_copy(table_hbm.at[gidx_v.at[j]], rows_v.at[buf], sem).wait()

        def _scale(j, buf):
            def _grp(g, carry):
                nv = norm_v[j, pl.ds(g * 16, 16)]
                for ee in range(16):
                    sc = nv[ee]
                    e = g * 16 + ee
                    for cc in range(_D // 16):
                        sl = pl.ds(cc * 16, 16)
                        rows_v[buf, e, sl] = rows_v[buf, e, sl] * sc
                return carry
            lax.fori_loop(0, _KB // 16, _grp, 0)

        def _scatter(j, buf):
            pltpu.sync_copy(rows_v.at[buf], accum_sh.at[dst_v.at[j]], add=True)

        def _superblock(sb, carry):
            g = wid * _NSB + sb
            pltpu.sync_copy(src_hbm.at[g], gidx_v)
            pltpu.sync_copy(r_hbm.at[g], r_v)
            pltpu.sync_copy(dst_hbm.at[g], dst_v)
            pltpu.sync_copy(norm_hbm.at[g], norm_v)

            def _gidx_row(j, carry2):
                for cc in range(_KB // 16):
                    sl = pl.ds(cc * 16, 16)
                    gidx_v[j, sl] = r_v[j, sl] * _N + gidx_v[j, sl]
                return carry2
            lax.fori_loop(0, _SB, _gidx_row, 0)

            _gather_start(0, 0, sem0)

            def _pair(jj, carry2):
                j0 = jj * 2
                j1 = j0 + 1
                _gather_start(j1, 1, sem1)
                _gather_wait(j0, 0, sem0)
                _scale(j0, 0)
                _scatter(j0, 0)
                _gather_start(j0 + 2, 0, sem0)
                _gather_wait(j1, 1, sem1)
                _scale(j1, 1)
                _scatter(j1, 1)
                return carry2
            lax.fori_loop(0, (_SB - 1) // 2, _pair, 0)

            _gather_wait(_SB - 1, 0, sem0)
            _scale(_SB - 1, 0)
            _scatter(_SB - 1, 0)
            return carry
        lax.fori_loop(0, _NSB, _superblock, 0)

        plsc.subcore_barrier()
        pltpu.sync_copy(accum_sh.at[pl.ds(s * _STRIPE, _STRIPE)],
                        out_hbm.at[c, pl.ds(s * _STRIPE, _STRIPE)])

    return k(table2d, src3, r3, dst3, norm3, zeros)


def kernel(h, edge_index, r, norm, W1, loop_w1, bias1, W2, loop_w2, bias2):
    src3 = edge_index[0].reshape(_NW * _NSB, _SB, _KB)
    dst3 = edge_index[1].reshape(_NW * _NSB, _SB, _KB)
    r3 = r.reshape(_NW * _NSB, _SB, _KB)
    norm3 = norm.reshape(_NW * _NSB, _SB, _KB)
    zeros = jnp.zeros((_NP, _D), jnp.float32)

    Wc1 = _expand_weights(W1, loop_w1)
    Wc2 = _expand_weights(W2, loop_w2)

    tmp1 = _table_from_h(h, Wc1)
    agg1 = _sc_gather_scatter(tmp1.reshape((_R + 1) * _N, _D),
                              src3, r3, dst3, norm3, zeros)
    tmp2 = _table_from_agg(agg1, tmp1, bias1, Wc2)
    agg2 = _sc_gather_scatter(tmp2.reshape((_R + 1) * _N, _D),
                              src3, r3, dst3, norm3, zeros)
    return _final(agg2, tmp2, bias2)
